# Initial kernel scaffold; baseline (speedup 1.0000x reference)
import jax
import jax.numpy as jnp
from jax.experimental import pallas as pl


def kernel(x, W_gates, b_gates, W_f, b_f, W_t, b_t, alpha, num):
    raise NotImplementedError("write your pallas kernel here")


# speedup vs baseline: 1.1054x; 1.1054x over previous
import functools

import jax
import jax.numpy as jnp
from jax.experimental import pallas as pl
from jax.experimental.pallas import tpu as pltpu

DIMS = 1024
E = 8
ROUTE_BT = 2048
GATE_BT = 1024


def _routing_body(x_ref, wt_ref, bt_ref, wf_ref, bf_ref, a_ref, f_ref):
    x = x_ref[...]
    lt = jnp.dot(x, wt_ref[...], preferred_element_type=jnp.float32) + bt_ref[...]
    iota = jax.lax.broadcasted_iota(jnp.int32, lt.shape, 1)
    m1 = jnp.max(lt, axis=-1, keepdims=True)
    i1 = jnp.min(jnp.where(lt == m1, iota, E), axis=-1, keepdims=True)
    masked = jnp.where(iota == i1, -jnp.inf, lt)
    m2 = jnp.max(masked, axis=-1, keepdims=True)
    i2 = jnp.min(jnp.where(masked == m2, iota, E), axis=-1, keepdims=True)
    t = jnp.exp(m2 - m1)
    w1 = 1.0 / (1.0 + t)
    w2 = t / (1.0 + t)
    type_ = jnp.where(iota == i1, w1, 0.0) + jnp.where(iota == i2, w2, 0.0)
    lf = jnp.dot(x, wf_ref[...], preferred_element_type=jnp.float32) + bf_ref[...]
    lf = lf - jnp.max(lf, axis=-1, keepdims=True)
    ef = jnp.exp(lf)
    feat = ef / jnp.sum(ef, axis=-1, keepdims=True)
    a = a_ref[0, 0]
    f_ref[...] = a * type_ + (1.0 - a) * feat


def _gates_body(x_ref, w_ref, b_ref, f_ref, o_ref):
    e = pl.program_id(1)
    z = jnp.dot(x_ref[...], w_ref[0], preferred_element_type=jnp.float32)
    g = jax.nn.sigmoid(z + b_ref[0])
    sel = jax.lax.broadcasted_iota(jnp.int32, f_ref.shape, 1) == e
    fcol = jnp.sum(jnp.where(sel, f_ref[...], 0.0), axis=1, keepdims=True)
    val = g * fcol

    @pl.when(e == 0)
    def _():
        o_ref[...] = val

    @pl.when(e > 0)
    def _():
        o_ref[...] += val


def kernel(x, W_gates, b_gates, W_f, b_f, W_t, b_t, alpha, num):
    B, S, D = x.shape
    M = B * S
    xf = x.reshape(M, D)
    a = jax.nn.sigmoid(alpha).reshape(1, 1).astype(jnp.float32)

    features = pl.pallas_call(
        _routing_body,
        grid=(M // ROUTE_BT,),
        in_specs=[
            pl.BlockSpec((ROUTE_BT, D), lambda t: (t, 0)),
            pl.BlockSpec((D, E), lambda t: (0, 0)),
            pl.BlockSpec((1, E), lambda t: (0, 0)),
            pl.BlockSpec((D, E), lambda t: (0, 0)),
            pl.BlockSpec((1, E), lambda t: (0, 0)),
            pl.BlockSpec((1, 1), lambda t: (0, 0)),
        ],
        out_specs=pl.BlockSpec((ROUTE_BT, E), lambda t: (t, 0)),
        out_shape=jax.ShapeDtypeStruct((M, E), jnp.float32),
    )(xf, W_t, b_t.reshape(1, E), W_f, b_f.reshape(1, E), a)

    xb = xf.astype(jnp.bfloat16)
    Wb = W_gates.astype(jnp.bfloat16)
    out = pl.pallas_call(
        _gates_body,
        grid=(M // GATE_BT, E),
        in_specs=[
            pl.BlockSpec((GATE_BT, D), lambda t, e: (t, 0)),
            pl.BlockSpec((1, D, D), lambda t, e: (e, 0, 0)),
            pl.BlockSpec((1, 1, D), lambda t, e: (e, 0, 0)),
            pl.BlockSpec((GATE_BT, E), lambda t, e: (t, 0)),
        ],
        out_specs=pl.BlockSpec((GATE_BT, D), lambda t, e: (t, 0)),
        out_shape=jax.ShapeDtypeStruct((M, D), jnp.float32),
        compiler_params=pltpu.CompilerParams(
            dimension_semantics=("parallel", "arbitrary"),
        ),
    )(xb, Wb, b_gates.reshape(E, 1, D), features)

    return out.reshape(B, S, D)


# speedup vs baseline: 1.4329x; 1.2962x over previous
import jax
import jax.numpy as jnp
from jax.experimental import pallas as pl
from jax.experimental.pallas import tpu as pltpu

DIMS = 1024
E = 8
BT = 1024


def _body(x_ref, w_ref, bg_ref, wt_ref, bt_ref, wf_ref, bf_ref, a_ref, o_ref):
    x = x_ref[...]
    lt = jnp.dot(x, wt_ref[...], preferred_element_type=jnp.float32) + bt_ref[...]
    iota = jax.lax.broadcasted_iota(jnp.int32, lt.shape, 1)
    m1 = jnp.max(lt, axis=-1, keepdims=True)
    i1 = jnp.min(jnp.where(lt == m1, iota, E), axis=-1, keepdims=True)
    masked = jnp.where(iota == i1, -jnp.inf, lt)
    m2 = jnp.max(masked, axis=-1, keepdims=True)
    i2 = jnp.min(jnp.where(masked == m2, iota, E), axis=-1, keepdims=True)
    t = jnp.exp(m2 - m1)
    w1 = 1.0 / (1.0 + t)
    w2 = t / (1.0 + t)
    type_ = jnp.where(iota == i1, w1, 0.0) + jnp.where(iota == i2, w2, 0.0)
    lf = jnp.dot(x, wf_ref[...], preferred_element_type=jnp.float32) + bf_ref[...]
    lf = lf - jnp.max(lf, axis=-1, keepdims=True)
    ef = jnp.exp(lf)
    feat = ef / jnp.sum(ef, axis=-1, keepdims=True)
    a = a_ref[0, 0]
    feats = a * type_ + (1.0 - a) * feat
    xb = x.astype(jnp.bfloat16)
    acc = None
    for e in range(E):
        z = jnp.dot(xb, w_ref[e], preferred_element_type=jnp.float32)
        g = jax.nn.sigmoid(z + bg_ref[e])
        v = g * feats[:, e:e + 1]
        acc = v if acc is None else acc + v
    o_ref[...] = acc


def kernel(x, W_gates, b_gates, W_f, b_f, W_t, b_t, alpha, num):
    B, S, D = x.shape
    M = B * S
    xf = x.reshape(M, D)
    a = jax.nn.sigmoid(alpha).reshape(1, 1).astype(jnp.float32)
    Wb = W_gates.astype(jnp.bfloat16)

    out = pl.pallas_call(
        _body,
        grid=(M // BT,),
        in_specs=[
            pl.BlockSpec((BT, D), lambda t: (t, 0)),
            pl.BlockSpec((E, D, D), lambda t: (0, 0, 0)),
            pl.BlockSpec((E, D), lambda t: (0, 0)),
            pl.BlockSpec((D, E), lambda t: (0, 0)),
            pl.BlockSpec((1, E), lambda t: (0, 0)),
            pl.BlockSpec((D, E), lambda t: (0, 0)),
            pl.BlockSpec((1, E), lambda t: (0, 0)),
            pl.BlockSpec((1, 1), lambda t: (0, 0)),
        ],
        out_specs=pl.BlockSpec((BT, D), lambda t: (t, 0)),
        out_shape=jax.ShapeDtypeStruct((M, D), jnp.float32),
        compiler_params=pltpu.CompilerParams(
            dimension_semantics=("arbitrary",),
        ),
    )(xf, Wb, b_gates, W_t, b_t.reshape(1, E), W_f, b_f.reshape(1, E), a)

    return out.reshape(B, S, D)


# speedup vs baseline: 1.5792x; 1.1022x over previous
import jax
import jax.numpy as jnp
from jax.experimental import pallas as pl
from jax.experimental.pallas import tpu as pltpu

DIMS = 1024
E = 8
BT = 1024


def _body(x_ref, w_ref, bg_ref, wc_ref, bc_ref, a_ref, o_ref):
    x = x_ref[...]
    lc = jnp.dot(x, wc_ref[...], preferred_element_type=jnp.float32) + bc_ref[...]
    lt = lc[:, :E]
    lf = lc[:, E:]
    iota = jax.lax.broadcasted_iota(jnp.int32, lt.shape, 1)
    m1 = jnp.max(lt, axis=-1, keepdims=True)
    i1 = jnp.min(jnp.where(lt == m1, iota, E), axis=-1, keepdims=True)
    masked = jnp.where(iota == i1, -jnp.inf, lt)
    m2 = jnp.max(masked, axis=-1, keepdims=True)
    i2 = jnp.min(jnp.where(masked == m2, iota, E), axis=-1, keepdims=True)
    t = jnp.exp(m2 - m1)
    w1 = 1.0 / (1.0 + t)
    w2 = t / (1.0 + t)
    type_ = jnp.where(iota == i1, w1, 0.0) + jnp.where(iota == i2, w2, 0.0)
    lf = lf - jnp.max(lf, axis=-1, keepdims=True)
    ef = jnp.exp(lf)
    feat = ef / jnp.sum(ef, axis=-1, keepdims=True)
    a = a_ref[0, 0]
    hfeats = (0.5 * a) * type_ + (0.5 * (1.0 - a)) * feat
    xb = (0.5 * x).astype(jnp.bfloat16)
    acc = jnp.sum(hfeats, axis=-1, keepdims=True)
    for e in range(E):
        z = jnp.dot(xb, w_ref[e], preferred_element_type=jnp.float32)
        th = jnp.tanh(z + bg_ref[e])
        acc = acc + th * hfeats[:, e:e + 1]
    o_ref[...] = acc


def kernel(x, W_gates, b_gates, W_f, b_f, W_t, b_t, alpha, num):
    B, S, D = x.shape
    M = B * S
    xf = x.reshape(M, D)
    a = jax.nn.sigmoid(alpha).reshape(1, 1).astype(jnp.float32)
    Wb = W_gates.astype(jnp.bfloat16)
    Wc = jnp.concatenate([W_t, W_f], axis=1)
    bc = jnp.concatenate([b_t, b_f]).reshape(1, 2 * E)
    hbg = 0.5 * b_gates

    out = pl.pallas_call(
        _body,
        grid=(M // BT,),
        in_specs=[
            pl.BlockSpec((BT, D), lambda t: (t, 0)),
            pl.BlockSpec((E, D, D), lambda t: (0, 0, 0)),
            pl.BlockSpec((E, D), lambda t: (0, 0)),
            pl.BlockSpec((D, 2 * E), lambda t: (0, 0)),
            pl.BlockSpec((1, 2 * E), lambda t: (0, 0)),
            pl.BlockSpec((1, 1), lambda t: (0, 0)),
        ],
        out_specs=pl.BlockSpec((BT, D), lambda t: (t, 0)),
        out_shape=jax.ShapeDtypeStruct((M, D), jnp.float32),
        compiler_params=pltpu.CompilerParams(
            dimension_semantics=("parallel",),
        ),
    )(xf, Wb, hbg, Wc, bc, a)

    return out.reshape(B, S, D)
